# baseline (speedup 1.0000x reference)
import functools

import jax
import jax.numpy as jnp
from jax import lax
from jax.experimental import pallas as pl
from jax.experimental.pallas import tpu as pltpu
from jax.experimental.pallas import tpu_sc as plsc

N_NODES = 10000
N_EDGES = 320000
D_FEAT = 128

NC = 2
NS = 16
LANES = 128
NW = NC * NS
EROWS = N_EDGES // LANES
WROWS = EROWS // NW
XTRA = EROWS - WROWS * NW
WEDGE = (WROWS + 1) * LANES
NR = 625
ACC = 10240
ZCHUNK = ACC // NS
GTAIL = N_NODES - (NS - 1) * ZCHUNK

_SC_PARAMS = pltpu.CompilerParams(
    use_tc_tiling_on_sc=False, needs_layout_passes=False)


def _ids():
    cid = lax.axis_index("c")
    sid = lax.axis_index("s")
    return cid, sid, sid * NC + cid


def _row0(wid):
    return jnp.where(wid < XTRA, (WROWS + 1) * wid, WROWS * wid + XTRA)


def _zero_init(zbuf, acc_sh, sid):
    def zfill(t, carry):
        zbuf[pl.ds(t * 16, 16)] = jnp.zeros((16,), jnp.float32)
        return carry

    lax.fori_loop(0, ZCHUNK // 16, zfill, 0)
    pltpu.sync_copy(zbuf, acc_sh.at[pl.ds(sid * ZCHUNK, ZCHUNK)])


def _stage_edges(edges_hbm, comp, buf, row0):
    pltpu.sync_copy(edges_hbm.at[comp, pl.ds(row0 * LANES, WROWS * LANES)],
                    buf.at[pl.ds(0, WROWS * LANES)])


def _stage_edges_xtra(edges_hbm, comp, buf, row0):
    pltpu.sync_copy(edges_hbm.at[comp, pl.ds((row0 + WROWS) * LANES, LANES)],
                    buf.at[pl.ds(WROWS * LANES, LANES)])


def _rsqrt_nr(d):
    i = plsc.bitcast(d, jnp.int32)
    i = jnp.int32(0x5F3759DF) - lax.shift_right_arithmetic(i, 1)
    y = plsc.bitcast(i, jnp.float32)
    for _ in range(3):
        y = y * (1.5 - 0.5 * d * y * y)
    return y


@functools.partial(
    pl.kernel,
    out_type=jax.ShapeDtypeStruct((NC, N_NODES), jnp.float32),
    mesh=plsc.VectorSubcoreMesh(core_axis_name="c", subcore_axis_name="s"),
    compiler_params=_SC_PARAMS,
    scratch_types=[
        pltpu.VMEM((WEDGE,), jnp.int32),
        pltpu.VMEM((LANES,), jnp.float32),
        pltpu.VMEM((ZCHUNK,), jnp.float32),
        pltpu.VMEM_SHARED((ACC,), jnp.float32),
        pltpu.SemaphoreType.DMA,
    ],
)
def _hist_kernel(edges_hbm, out_hbm, dst_v, ones_v, zbuf, acc_sh, sem):
    cid, sid, wid = _ids()
    row0 = _row0(wid)

    _stage_edges(edges_hbm, 1, dst_v, row0)

    @pl.when(wid < XTRA)
    def _():
        _stage_edges_xtra(edges_hbm, 1, dst_v, row0)

    for t in range(LANES // 16):
        ones_v[pl.ds(t * 16, 16)] = jnp.ones((16,), jnp.float32)

    _zero_init(zbuf, acc_sh, sid)
    plsc.subcore_barrier()

    def issue(j, carry):
        pltpu.async_copy(ones_v, acc_sh.at[dst_v.at[pl.ds(j * LANES, LANES)]],
                         sem, add=True)
        return carry

    lax.fori_loop(0, WROWS, issue, 0)

    @pl.when(wid < XTRA)
    def _():
        issue(WROWS, 0)

    def drain(j, carry):
        pltpu.make_async_copy(ones_v, acc_sh.at[dst_v.at[pl.ds(0, LANES)]],
                              sem).wait()
        return carry

    lax.fori_loop(0, WROWS, drain, 0)

    @pl.when(wid < XTRA)
    def _():
        drain(0, 0)

    plsc.subcore_barrier()

    @pl.when(sid == 0)
    def _():
        pltpu.sync_copy(acc_sh.at[pl.ds(0, N_NODES)], out_hbm.at[cid])


@functools.partial(
    pl.kernel,
    out_type=jax.ShapeDtypeStruct((NC, N_NODES), jnp.float32),
    mesh=plsc.VectorSubcoreMesh(core_axis_name="c", subcore_axis_name="s"),
    compiler_params=_SC_PARAMS,
    scratch_types=[
        pltpu.VMEM((ACC,), jnp.float32),
        pltpu.VMEM((WEDGE,), jnp.int32),
        pltpu.VMEM((WEDGE,), jnp.int32),
        pltpu.VMEM((WEDGE,), jnp.float32),
        pltpu.VMEM((ZCHUNK,), jnp.float32),
        pltpu.VMEM((ZCHUNK,), jnp.float32),
        pltpu.VMEM((ZCHUNK,), jnp.float32),
        pltpu.VMEM((ZCHUNK,), jnp.float32),
        pltpu.VMEM_SHARED((ACC,), jnp.float32),
        pltpu.VMEM_SHARED((ACC,), jnp.float32),
        pltpu.SemaphoreType.DMA,
        pltpu.SemaphoreType.DMA,
    ],
)
def _msg_kernel(edges_hbm, deg2_hbm, h_hbm, out_hbm,
                g_v, src_v, dst_v, vals_v, d0_v, d1_v, zbuf, dis_v,
                acc_sh, g_sh, sem, gsem):
    cid, sid, wid = _ids()
    row0 = _row0(wid)

    c_src = pltpu.async_copy(
        edges_hbm.at[0, pl.ds(row0 * LANES, WROWS * LANES)],
        src_v.at[pl.ds(0, WROWS * LANES)], gsem)
    c_dst = pltpu.async_copy(
        edges_hbm.at[1, pl.ds(row0 * LANES, WROWS * LANES)],
        dst_v.at[pl.ds(0, WROWS * LANES)], gsem)

    @pl.when(wid < XTRA)
    def _():
        _stage_edges_xtra(edges_hbm, 0, src_v, row0)
        _stage_edges_xtra(edges_hbm, 1, dst_v, row0)

    node0 = sid * ZCHUNK

    @pl.when(sid < NS - 1)
    def _():
        pltpu.sync_copy(deg2_hbm.at[0, pl.ds(node0, ZCHUNK)], d0_v)
        pltpu.sync_copy(deg2_hbm.at[1, pl.ds(node0, ZCHUNK)], d1_v)
        pltpu.sync_copy(h_hbm.at[pl.ds(node0, ZCHUNK)], zbuf)

    @pl.when(sid == NS - 1)
    def _():
        pltpu.sync_copy(deg2_hbm.at[0, pl.ds(node0, GTAIL)],
                        d0_v.at[pl.ds(0, GTAIL)])
        pltpu.sync_copy(deg2_hbm.at[1, pl.ds(node0, GTAIL)],
                        d1_v.at[pl.ds(0, GTAIL)])
        pltpu.sync_copy(h_hbm.at[pl.ds(node0, GTAIL)],
                        zbuf.at[pl.ds(0, GTAIL)])

    def gfill(t, carry):
        sl = pl.ds(t * 16, 16)
        deg = d0_v[sl] + d1_v[sl] + 1.0
        r = _rsqrt_nr(deg)
        dis_v[sl] = r
        d0_v[sl] = r * zbuf[sl]
        return carry

    lax.fori_loop(0, ZCHUNK // 16, gfill, 0)

    @pl.when(sid < NS - 1)
    def _():
        pltpu.sync_copy(d0_v, g_sh.at[pl.ds(node0, ZCHUNK)])

    @pl.when(sid == NS - 1)
    def _():
        pltpu.sync_copy(d0_v.at[pl.ds(0, GTAIL)],
                        g_sh.at[pl.ds(node0, GTAIL)])

    _zero_init(zbuf, acc_sh, sid)
    plsc.subcore_barrier()
    pltpu.sync_copy(g_sh.at[pl.ds(0, N_NODES)], g_v.at[pl.ds(0, N_NODES)])
    c_src.wait()
    c_dst.wait()

    def row_fn(j):
        base = j * LANES
        for t in range(LANES // 16):
            idx = src_v[pl.ds(base + t * 16, 16)]
            vals_v[pl.ds(base + t * 16, 16)] = plsc.load_gather(g_v, [idx])
        pltpu.async_copy(vals_v.at[pl.ds(base, LANES)],
                         acc_sh.at[dst_v.at[pl.ds(base, LANES)]],
                         sem, add=True)

    @plsc.parallel_loop(0, WROWS, step=1, unroll=2)
    def _(j):
        row_fn(j)

    @pl.when(wid < XTRA)
    def _():
        row_fn(WROWS)

    def drain(j, carry):
        pltpu.make_async_copy(vals_v.at[pl.ds(0, LANES)],
                              acc_sh.at[dst_v.at[pl.ds(0, LANES)]],
                              sem).wait()
        return carry

    lax.fori_loop(0, WROWS, drain, 0)

    @pl.when(wid < XTRA)
    def _():
        drain(0, 0)

    plsc.subcore_barrier()

    @pl.when(sid < NS - 1)
    def _():
        pltpu.sync_copy(acc_sh.at[pl.ds(node0, ZCHUNK)], d1_v)

    @pl.when(sid == NS - 1)
    def _():
        pltpu.sync_copy(acc_sh.at[pl.ds(node0, GTAIL)],
                        d1_v.at[pl.ds(0, GTAIL)])

    def ofill(t, carry):
        sl = pl.ds(t * 16, 16)
        d1_v[sl] = dis_v[sl] * (0.5 * d0_v[sl] + d1_v[sl])
        return carry

    lax.fori_loop(0, ZCHUNK // 16, ofill, 0)

    @pl.when(sid < NS - 1)
    def _():
        pltpu.sync_copy(d1_v, out_hbm.at[cid, pl.ds(node0, ZCHUNK)])

    @pl.when(sid == NS - 1)
    def _():
        pltpu.sync_copy(d1_v.at[pl.ds(0, GTAIL)],
                        out_hbm.at[cid, pl.ds(node0, GTAIL)])


def _matvec_body(x3_ref, w_ref, h_ref):
    h_ref[...] = jnp.sum(x3_ref[...] * w_ref[...], axis=2)


_matvec_call = pl.pallas_call(
    _matvec_body,
    out_shape=jax.ShapeDtypeStruct((NR, 16), jnp.float32),
)


def kernel(x, edge_index, W):
    edges = edge_index.astype(jnp.int32)
    x3 = x.reshape(NR, 16, D_FEAT)
    w3 = W.reshape(1, 1, D_FEAT)

    h = _matvec_call(x3, w3)
    deg2 = _hist_kernel(edges)
    out2 = _msg_kernel(edges, deg2, h.reshape(N_NODES))
    return out2[0] + out2[1]


# speedup vs baseline: 1.0279x; 1.0279x over previous
import functools

import jax
import jax.numpy as jnp
from jax import lax
from jax.experimental import pallas as pl
from jax.experimental.pallas import tpu as pltpu
from jax.experimental.pallas import tpu_sc as plsc

N_NODES = 10000
N_EDGES = 320000
D_FEAT = 128

NC = 2
NS = 16
LANES = 128
NW = NC * NS
EROWS = N_EDGES // LANES
WROWS = EROWS // NW
XTRA = EROWS - WROWS * NW
WEDGE = (WROWS + 1) * LANES
NR = 625
ACC = 10240
ZCHUNK = ACC // NS
GTAIL = N_NODES - (NS - 1) * ZCHUNK

_SC_PARAMS = pltpu.CompilerParams(
    use_tc_tiling_on_sc=False, needs_layout_passes=False)


def _ids():
    cid = lax.axis_index("c")
    sid = lax.axis_index("s")
    return cid, sid, sid * NC + cid


def _row0(wid):
    return jnp.where(wid < XTRA, (WROWS + 1) * wid, WROWS * wid + XTRA)


def _zero_init(zbuf, acc_sh, sid):
    def zfill(t, carry):
        zbuf[pl.ds(t * 16, 16)] = jnp.zeros((16,), jnp.float32)
        return carry

    lax.fori_loop(0, ZCHUNK // 16, zfill, 0)
    pltpu.sync_copy(zbuf, acc_sh.at[pl.ds(sid * ZCHUNK, ZCHUNK)])


def _stage_edges(edges_hbm, comp, buf, row0):
    pltpu.sync_copy(edges_hbm.at[comp, pl.ds(row0 * LANES, WROWS * LANES)],
                    buf.at[pl.ds(0, WROWS * LANES)])


def _stage_edges_xtra(edges_hbm, comp, buf, row0):
    pltpu.sync_copy(edges_hbm.at[comp, pl.ds((row0 + WROWS) * LANES, LANES)],
                    buf.at[pl.ds(WROWS * LANES, LANES)])


def _rsqrt_nr(d):
    i = plsc.bitcast(d, jnp.int32)
    i = jnp.int32(0x5F3759DF) - lax.shift_right_arithmetic(i, 1)
    y = plsc.bitcast(i, jnp.float32)
    for _ in range(3):
        y = y * (1.5 - 0.5 * d * y * y)
    return y


@functools.partial(
    pl.kernel,
    out_type=jax.ShapeDtypeStruct((NC, N_NODES), jnp.float32),
    mesh=plsc.VectorSubcoreMesh(core_axis_name="c", subcore_axis_name="s"),
    compiler_params=_SC_PARAMS,
    scratch_types=[
        pltpu.VMEM((WEDGE,), jnp.int32),
        pltpu.VMEM((LANES,), jnp.float32),
        pltpu.VMEM((ZCHUNK,), jnp.float32),
        pltpu.VMEM_SHARED((ACC,), jnp.float32),
        pltpu.SemaphoreType.DMA,
        pltpu.SemaphoreType.DMA,
    ],
)
def _hist_kernel(edges_hbm, out_hbm, dst_v, ones_v, zbuf, acc_sh, sem, ssem):
    cid, sid, wid = _ids()
    row0 = _row0(wid)

    c_dst = pltpu.async_copy(
        edges_hbm.at[1, pl.ds(row0 * LANES, WROWS * LANES)],
        dst_v.at[pl.ds(0, WROWS * LANES)], ssem)

    @pl.when(wid < XTRA)
    def _():
        _stage_edges_xtra(edges_hbm, 1, dst_v, row0)

    for t in range(LANES // 16):
        ones_v[pl.ds(t * 16, 16)] = jnp.ones((16,), jnp.float32)

    _zero_init(zbuf, acc_sh, sid)
    c_dst.wait()
    plsc.subcore_barrier()

    def issue(j, carry):
        pltpu.async_copy(ones_v, acc_sh.at[dst_v.at[pl.ds(j * LANES, LANES)]],
                         sem, add=True)
        return carry

    lax.fori_loop(0, WROWS, issue, 0)

    @pl.when(wid < XTRA)
    def _():
        issue(WROWS, 0)

    def drain(j, carry):
        pltpu.make_async_copy(ones_v, acc_sh.at[dst_v.at[pl.ds(0, LANES)]],
                              sem).wait()
        return carry

    lax.fori_loop(0, WROWS, drain, 0)

    @pl.when(wid < XTRA)
    def _():
        drain(0, 0)

    plsc.subcore_barrier()

    @pl.when(sid == 0)
    def _():
        pltpu.sync_copy(acc_sh.at[pl.ds(0, N_NODES)], out_hbm.at[cid])


@functools.partial(
    pl.kernel,
    out_type=jax.ShapeDtypeStruct((NC, N_NODES), jnp.float32),
    mesh=plsc.VectorSubcoreMesh(core_axis_name="c", subcore_axis_name="s"),
    compiler_params=_SC_PARAMS,
    scratch_types=[
        pltpu.VMEM((ACC,), jnp.float32),
        pltpu.VMEM((WEDGE,), jnp.int32),
        pltpu.VMEM((WEDGE,), jnp.int32),
        pltpu.VMEM((WEDGE,), jnp.float32),
        pltpu.VMEM((ZCHUNK,), jnp.float32),
        pltpu.VMEM((ZCHUNK,), jnp.float32),
        pltpu.VMEM((ZCHUNK,), jnp.float32),
        pltpu.VMEM((ZCHUNK,), jnp.float32),
        pltpu.VMEM_SHARED((ACC,), jnp.float32),
        pltpu.VMEM_SHARED((ACC,), jnp.float32),
        pltpu.SemaphoreType.DMA,
        pltpu.SemaphoreType.DMA,
    ],
)
def _msg_kernel(edges_hbm, deg2_hbm, h_hbm, out_hbm,
                g_v, src_v, dst_v, vals_v, d0_v, d1_v, zbuf, dis_v,
                acc_sh, g_sh, sem, gsem):
    cid, sid, wid = _ids()
    row0 = _row0(wid)

    c_src = pltpu.async_copy(
        edges_hbm.at[0, pl.ds(row0 * LANES, WROWS * LANES)],
        src_v.at[pl.ds(0, WROWS * LANES)], gsem)
    c_dst = pltpu.async_copy(
        edges_hbm.at[1, pl.ds(row0 * LANES, WROWS * LANES)],
        dst_v.at[pl.ds(0, WROWS * LANES)], gsem)

    @pl.when(wid < XTRA)
    def _():
        _stage_edges_xtra(edges_hbm, 0, src_v, row0)
        _stage_edges_xtra(edges_hbm, 1, dst_v, row0)

    node0 = sid * ZCHUNK

    @pl.when(sid < NS - 1)
    def _():
        pltpu.sync_copy(deg2_hbm.at[0, pl.ds(node0, ZCHUNK)], d0_v)
        pltpu.sync_copy(deg2_hbm.at[1, pl.ds(node0, ZCHUNK)], d1_v)
        pltpu.sync_copy(h_hbm.at[pl.ds(node0, ZCHUNK)], zbuf)

    @pl.when(sid == NS - 1)
    def _():
        pltpu.sync_copy(deg2_hbm.at[0, pl.ds(node0, GTAIL)],
                        d0_v.at[pl.ds(0, GTAIL)])
        pltpu.sync_copy(deg2_hbm.at[1, pl.ds(node0, GTAIL)],
                        d1_v.at[pl.ds(0, GTAIL)])
        pltpu.sync_copy(h_hbm.at[pl.ds(node0, GTAIL)],
                        zbuf.at[pl.ds(0, GTAIL)])

    def gfill(t, carry):
        sl = pl.ds(t * 16, 16)
        deg = d0_v[sl] + d1_v[sl] + 1.0
        r = _rsqrt_nr(deg)
        dis_v[sl] = r
        d0_v[sl] = r * zbuf[sl]
        return carry

    lax.fori_loop(0, ZCHUNK // 16, gfill, 0)

    @pl.when(sid < NS - 1)
    def _():
        pltpu.sync_copy(d0_v, g_sh.at[pl.ds(node0, ZCHUNK)])

    @pl.when(sid == NS - 1)
    def _():
        pltpu.sync_copy(d0_v.at[pl.ds(0, GTAIL)],
                        g_sh.at[pl.ds(node0, GTAIL)])

    _zero_init(zbuf, acc_sh, sid)
    plsc.subcore_barrier()
    pltpu.sync_copy(g_sh.at[pl.ds(0, N_NODES)], g_v.at[pl.ds(0, N_NODES)])
    c_src.wait()
    c_dst.wait()

    def row_fn(j, carry):
        base = j * LANES
        for t in range(LANES // 16):
            idx = src_v[pl.ds(base + t * 16, 16)]
            vals_v[pl.ds(base + t * 16, 16)] = plsc.load_gather(g_v, [idx])
        pltpu.async_copy(vals_v.at[pl.ds(base, LANES)],
                         acc_sh.at[dst_v.at[pl.ds(base, LANES)]],
                         sem, add=True)
        return carry

    lax.fori_loop(0, WROWS, row_fn, 0)

    @pl.when(wid < XTRA)
    def _():
        row_fn(WROWS, 0)

    def drain(j, carry):
        pltpu.make_async_copy(vals_v.at[pl.ds(0, LANES)],
                              acc_sh.at[dst_v.at[pl.ds(0, LANES)]],
                              sem).wait()
        return carry

    lax.fori_loop(0, WROWS, drain, 0)

    @pl.when(wid < XTRA)
    def _():
        drain(0, 0)

    plsc.subcore_barrier()

    @pl.when(sid < NS - 1)
    def _():
        pltpu.sync_copy(acc_sh.at[pl.ds(node0, ZCHUNK)], d1_v)

    @pl.when(sid == NS - 1)
    def _():
        pltpu.sync_copy(acc_sh.at[pl.ds(node0, GTAIL)],
                        d1_v.at[pl.ds(0, GTAIL)])

    def ofill(t, carry):
        sl = pl.ds(t * 16, 16)
        d1_v[sl] = dis_v[sl] * (0.5 * d0_v[sl] + d1_v[sl])
        return carry

    lax.fori_loop(0, ZCHUNK // 16, ofill, 0)

    @pl.when(sid < NS - 1)
    def _():
        pltpu.sync_copy(d1_v, out_hbm.at[cid, pl.ds(node0, ZCHUNK)])

    @pl.when(sid == NS - 1)
    def _():
        pltpu.sync_copy(d1_v.at[pl.ds(0, GTAIL)],
                        out_hbm.at[cid, pl.ds(node0, GTAIL)])


def _matvec_body(x3_ref, w_ref, h_ref):
    h_ref[...] = jnp.sum(x3_ref[...] * w_ref[...], axis=2)


_matvec_call = pl.pallas_call(
    _matvec_body,
    out_shape=jax.ShapeDtypeStruct((NR, 16), jnp.float32),
)


def kernel(x, edge_index, W):
    edges = edge_index.astype(jnp.int32)
    x3 = x.reshape(NR, 16, D_FEAT)
    w3 = W.reshape(1, 1, D_FEAT)

    h = _matvec_call(x3, w3)
    deg2 = _hist_kernel(edges)
    out2 = _msg_kernel(edges, deg2, h.reshape(N_NODES))
    return out2[0] + out2[1]


# speedup vs baseline: 1.0284x; 1.0005x over previous
import functools

import jax
import jax.numpy as jnp
from jax import lax
from jax.experimental import pallas as pl
from jax.experimental.pallas import tpu as pltpu
from jax.experimental.pallas import tpu_sc as plsc

N_NODES = 10000
N_EDGES = 320000
D_FEAT = 128

NC = 2
NS = 16
LANES = 128
NW = NC * NS
EROWS = N_EDGES // LANES
WROWS = EROWS // NW
XTRA = EROWS - WROWS * NW
WEDGE = (WROWS + 1) * LANES
NR = 625
ACC = 10240
ZCHUNK = ACC // NS
GTAIL = N_NODES - (NS - 1) * ZCHUNK

_SC_PARAMS = pltpu.CompilerParams(
    use_tc_tiling_on_sc=False, needs_layout_passes=False)


def _ids():
    cid = lax.axis_index("c")
    sid = lax.axis_index("s")
    return cid, sid, sid * NC + cid


def _row0(wid):
    return jnp.where(wid < XTRA, (WROWS + 1) * wid, WROWS * wid + XTRA)


def _zero_init(zbuf, acc_sh, sid):
    def zfill(t, carry):
        zbuf[pl.ds(t * 16, 16)] = jnp.zeros((16,), jnp.float32)
        return carry

    lax.fori_loop(0, ZCHUNK // 16, zfill, 0)
    pltpu.sync_copy(zbuf, acc_sh.at[pl.ds(sid * ZCHUNK, ZCHUNK)])


def _stage_edges_xtra(edges_hbm, comp, buf, row0):
    pltpu.sync_copy(edges_hbm.at[comp, pl.ds((row0 + WROWS) * LANES, LANES)],
                    buf.at[pl.ds(WROWS * LANES, LANES)])


def _rsqrt_nr(d):
    i = plsc.bitcast(d, jnp.int32)
    i = jnp.int32(0x5F3759DF) - lax.shift_right_arithmetic(i, 1)
    y = plsc.bitcast(i, jnp.float32)
    for _ in range(3):
        y = y * (1.5 - 0.5 * d * y * y)
    return y


@functools.partial(
    pl.kernel,
    out_type=jax.ShapeDtypeStruct((NC, N_NODES), jnp.float32),
    mesh=plsc.VectorSubcoreMesh(core_axis_name="c", subcore_axis_name="s"),
    compiler_params=_SC_PARAMS,
    scratch_types=[
        pltpu.VMEM((WEDGE,), jnp.int32),
        pltpu.VMEM((LANES,), jnp.float32),
        pltpu.VMEM((ZCHUNK,), jnp.float32),
        pltpu.VMEM_SHARED((ACC,), jnp.float32),
        pltpu.SemaphoreType.DMA,
        pltpu.SemaphoreType.DMA,
    ],
)
def _hist_kernel(edges_hbm, out_hbm, dst_v, ones_v, zbuf, acc_sh, sem, ssem):
    cid, sid, wid = _ids()
    row0 = _row0(wid)

    c_dst = pltpu.async_copy(
        edges_hbm.at[1, pl.ds(row0 * LANES, WROWS * LANES)],
        dst_v.at[pl.ds(0, WROWS * LANES)], ssem)

    @pl.when(wid < XTRA)
    def _():
        _stage_edges_xtra(edges_hbm, 1, dst_v, row0)

    for t in range(LANES // 16):
        ones_v[pl.ds(t * 16, 16)] = jnp.ones((16,), jnp.float32)

    _zero_init(zbuf, acc_sh, sid)
    c_dst.wait()
    plsc.subcore_barrier()

    def issue(j, carry):
        pltpu.async_copy(ones_v, acc_sh.at[dst_v.at[pl.ds(j * LANES, LANES)]],
                         sem, add=True)
        return carry

    lax.fori_loop(0, WROWS, issue, 0)

    @pl.when(wid < XTRA)
    def _():
        issue(WROWS, 0)

    def drain(j, carry):
        pltpu.make_async_copy(ones_v, acc_sh.at[dst_v.at[pl.ds(0, LANES)]],
                              sem).wait()
        return carry

    lax.fori_loop(0, WROWS, drain, 0)

    @pl.when(wid < XTRA)
    def _():
        drain(0, 0)

    plsc.subcore_barrier()

    @pl.when(sid == 0)
    def _():
        pltpu.sync_copy(acc_sh.at[pl.ds(0, N_NODES)], out_hbm.at[cid])


@functools.partial(
    pl.kernel,
    out_type=jax.ShapeDtypeStruct((NC, N_NODES), jnp.float32),
    mesh=plsc.VectorSubcoreMesh(core_axis_name="c", subcore_axis_name="s"),
    compiler_params=_SC_PARAMS,
    scratch_types=[
        pltpu.VMEM((ACC,), jnp.float32),
        pltpu.VMEM((WEDGE,), jnp.int32),
        pltpu.VMEM((WEDGE,), jnp.int32),
        pltpu.VMEM((WEDGE,), jnp.float32),
        pltpu.VMEM((ZCHUNK,), jnp.float32),
        pltpu.VMEM((ZCHUNK,), jnp.float32),
        pltpu.VMEM((ZCHUNK,), jnp.float32),
        pltpu.VMEM((ZCHUNK,), jnp.float32),
        pltpu.VMEM_SHARED((ACC,), jnp.float32),
        pltpu.VMEM_SHARED((ACC,), jnp.float32),
        pltpu.SemaphoreType.DMA,
        pltpu.SemaphoreType.DMA,
    ],
)
def _msg_kernel(edges_hbm, deg2_hbm, h_hbm, out_hbm,
                g_v, src_v, dst_v, vals_v, d0_v, d1_v, zbuf, dis_v,
                acc_sh, g_sh, sem, gsem):
    cid, sid, wid = _ids()
    row0 = _row0(wid)

    c_src = pltpu.async_copy(
        edges_hbm.at[0, pl.ds(row0 * LANES, WROWS * LANES)],
        src_v.at[pl.ds(0, WROWS * LANES)], gsem)
    c_dst = pltpu.async_copy(
        edges_hbm.at[1, pl.ds(row0 * LANES, WROWS * LANES)],
        dst_v.at[pl.ds(0, WROWS * LANES)], gsem)

    @pl.when(wid < XTRA)
    def _():
        _stage_edges_xtra(edges_hbm, 0, src_v, row0)
        _stage_edges_xtra(edges_hbm, 1, dst_v, row0)

    node0 = sid * ZCHUNK

    @pl.when(sid < NS - 1)
    def _():
        pltpu.sync_copy(deg2_hbm.at[0, pl.ds(node0, ZCHUNK)], d0_v)
        pltpu.sync_copy(deg2_hbm.at[1, pl.ds(node0, ZCHUNK)], d1_v)
        pltpu.sync_copy(h_hbm.at[pl.ds(node0, ZCHUNK)], zbuf)

    @pl.when(sid == NS - 1)
    def _():
        pltpu.sync_copy(deg2_hbm.at[0, pl.ds(node0, GTAIL)],
                        d0_v.at[pl.ds(0, GTAIL)])
        pltpu.sync_copy(deg2_hbm.at[1, pl.ds(node0, GTAIL)],
                        d1_v.at[pl.ds(0, GTAIL)])
        pltpu.sync_copy(h_hbm.at[pl.ds(node0, GTAIL)],
                        zbuf.at[pl.ds(0, GTAIL)])

    def gfill(t, carry):
        sl = pl.ds(t * 16, 16)
        deg = d0_v[sl] + d1_v[sl] + 1.0
        r = _rsqrt_nr(deg)
        dis_v[sl] = r
        d0_v[sl] = r * zbuf[sl]
        return carry

    lax.fori_loop(0, ZCHUNK // 16, gfill, 0)

    @pl.when(sid < NS - 1)
    def _():
        pltpu.sync_copy(d0_v, g_sh.at[pl.ds(node0, ZCHUNK)])

    @pl.when(sid == NS - 1)
    def _():
        pltpu.sync_copy(d0_v.at[pl.ds(0, GTAIL)],
                        g_sh.at[pl.ds(node0, GTAIL)])

    _zero_init(zbuf, acc_sh, sid)
    plsc.subcore_barrier()
    pltpu.sync_copy(g_sh.at[pl.ds(0, N_NODES)], g_v.at[pl.ds(0, N_NODES)])
    c_src.wait()
    c_dst.wait()

    def row_fn(j, carry):
        base = j * LANES
        for t in range(LANES // 16):
            idx = src_v[pl.ds(base + t * 16, 16)]
            vals_v[pl.ds(base + t * 16, 16)] = plsc.load_gather(g_v, [idx])
        pltpu.async_copy(vals_v.at[pl.ds(base, LANES)],
                         acc_sh.at[dst_v.at[pl.ds(base, LANES)]],
                         sem, add=True)
        return carry

    lax.fori_loop(0, WROWS, row_fn, 0)

    @pl.when(wid < XTRA)
    def _():
        row_fn(WROWS, 0)

    def drain(j, carry):
        pltpu.make_async_copy(vals_v.at[pl.ds(0, LANES)],
                              acc_sh.at[dst_v.at[pl.ds(0, LANES)]],
                              sem).wait()
        return carry

    lax.fori_loop(0, WROWS, drain, 0)

    @pl.when(wid < XTRA)
    def _():
        drain(0, 0)

    plsc.subcore_barrier()

    @pl.when(sid < NS - 1)
    def _():
        pltpu.sync_copy(acc_sh.at[pl.ds(node0, ZCHUNK)], d1_v)

    @pl.when(sid == NS - 1)
    def _():
        pltpu.sync_copy(acc_sh.at[pl.ds(node0, GTAIL)],
                        d1_v.at[pl.ds(0, GTAIL)])

    def ofill(t, carry):
        sl = pl.ds(t * 16, 16)
        d1_v[sl] = dis_v[sl] * (0.5 * d0_v[sl] + d1_v[sl])
        return carry

    lax.fori_loop(0, ZCHUNK // 16, ofill, 0)

    @pl.when(sid < NS - 1)
    def _():
        pltpu.sync_copy(d1_v, out_hbm.at[cid, pl.ds(node0, ZCHUNK)])

    @pl.when(sid == NS - 1)
    def _():
        pltpu.sync_copy(d1_v.at[pl.ds(0, GTAIL)],
                        out_hbm.at[cid, pl.ds(node0, GTAIL)])


def _matvec_body(x3_ref, w_ref, h_ref):
    h_ref[...] = jnp.sum(x3_ref[...] * w_ref[...], axis=2)


_matvec_call = pl.pallas_call(
    _matvec_body,
    out_shape=jax.ShapeDtypeStruct((NR, 16), jnp.float32),
)


def kernel(x, edge_index, W):
    edges = edge_index.astype(jnp.int32)
    x3 = x.reshape(NR, 16, D_FEAT)
    w3 = W.reshape(1, 1, D_FEAT)

    h = _matvec_call(x3, w3)
    deg2 = _hist_kernel(edges)
    out2 = _msg_kernel(edges, deg2, h.reshape(N_NODES))
    return out2[0] + out2[1]
